# Initial kernel scaffold; baseline (speedup 1.0000x reference)
import functools

import jax
import jax.numpy as jnp
from jax import lax
from jax.experimental import pallas as pl
from jax.experimental.pallas import tpu as pltpu
from jax.experimental.pallas import tpu_sc as plsc

N = 16384
BATCH = 256
NCHUNK = 4
CW = BATCH // NCHUNK
NCORES = 2
NSUB = 16
CPC = NCHUNK // NCORES
K = 128
ROWS_PER_TILE = N // NSUB


def _spmm_body(nblk, xc, rows, cols, vals, out, idxc, idxr, valv, gbuf, zbuf,
               acc, sem):
    c = lax.axis_index("c")
    s = lax.axis_index("s")
    per_tile = nblk * K

    zero16 = jnp.zeros((16,), jnp.float32)

    def zrow(i, carry):
        for w in range(CW // 16):
            zbuf[i, pl.ds(w * 16, 16)] = zero16
        return carry

    lax.fori_loop(0, K, zrow, 0)

    for jl in range(CPC):
        j = c * CPC + jl
        for z in range(ROWS_PER_TILE // K):
            pltpu.sync_copy(zbuf, acc.at[pl.ds(s * ROWS_PER_TILE + z * K, K)])
        plsc.subcore_barrier()

        col_off = j * N

        def blk(b, carry):
            base = s * per_tile + b * K
            pltpu.sync_copy(cols.at[pl.ds(base, K)], idxc)
            pltpu.sync_copy(rows.at[pl.ds(base, K)], idxr)
            pltpu.sync_copy(vals.at[pl.ds(base, K)], valv)
            for t in range(K // 16):
                idxc[pl.ds(t * 16, 16)] = idxc[pl.ds(t * 16, 16)] + col_off
            pltpu.async_copy(xc.at[idxc], gbuf, sem).wait()

            def rowscale(i, rc):
                v = valv[i]
                for w in range(CW // 16):
                    gbuf[i, pl.ds(w * 16, 16)] = gbuf[i, pl.ds(w * 16, 16)] * v
                return rc

            lax.fori_loop(0, K, rowscale, 0)
            pltpu.sync_copy(gbuf, acc.at[idxr], add=True)
            return carry

        lax.fori_loop(0, nblk, blk, 0)
        plsc.subcore_barrier()
        pltpu.sync_copy(acc.at[pl.ds(s * ROWS_PER_TILE, ROWS_PER_TILE)],
                        out.at[j, pl.ds(s * ROWS_PER_TILE, ROWS_PER_TILE)])
        plsc.subcore_barrier()


def kernel(X, S_rows, S_cols, S_vals):
    nnz = S_rows.shape[0]
    per_tile = -(-nnz // NSUB)
    per_tile = -(-per_tile // K) * K
    nblk = per_tile // K
    pad = per_tile * NSUB - nnz
    rows_p = jnp.pad(S_rows, (0, pad))
    cols_p = jnp.pad(S_cols, (0, pad))
    vals_p = jnp.pad(S_vals, (0, pad))
    xc = X.reshape(NCHUNK, CW, N).transpose(0, 2, 1).reshape(NCHUNK * N, CW)

    mesh = plsc.VectorSubcoreMesh(core_axis_name="c", subcore_axis_name="s",
                                  num_cores=NCORES, num_subcores=NSUB)
    f = pl.kernel(
        functools.partial(_spmm_body, nblk),
        out_type=jax.ShapeDtypeStruct((NCHUNK, N, CW), jnp.float32),
        mesh=mesh,
        scratch_types=[
            pltpu.VMEM((K,), jnp.int32),
            pltpu.VMEM((K,), jnp.int32),
            pltpu.VMEM((K,), jnp.float32),
            pltpu.VMEM((K, CW), jnp.float32),
            pltpu.VMEM((K, CW), jnp.float32),
            pltpu.VMEM_SHARED((N, CW), jnp.float32),
            pltpu.SemaphoreType.DMA,
        ],
    )
    out_c = f(xc, rows_p, cols_p, vals_p)
    return out_c.transpose(0, 2, 1).reshape(BATCH, N)


# speedup vs baseline: 2.9869x; 2.9869x over previous
import functools

import jax
import jax.numpy as jnp
from jax import lax
from jax.experimental import pallas as pl
from jax.experimental.pallas import tpu as pltpu
from jax.experimental.pallas import tpu_sc as plsc

N = 16384
BATCH = 256
NCHUNK = 4
CW = BATCH // NCHUNK
NCORES = 2
NSUB = 16
CPC = NCHUNK // NCORES
K = 128
ROWS_PER_TILE = N // NSUB


def _spmm_body(nblk, xc, rows, cols, vals, out, idxc, idxr, valv, gbuf, zbuf,
               acc, sem):
    c = lax.axis_index("c")
    s = lax.axis_index("s")
    per_tile = nblk * K

    zero16 = jnp.zeros((16,), jnp.float32)

    def zrow(i, carry):
        for w in range(CW // 16):
            zbuf[i, pl.ds(w * 16, 16)] = zero16
        return carry

    lax.fori_loop(0, K, zrow, 0)

    for jl in range(CPC):
        j = c * CPC + jl
        for z in range(ROWS_PER_TILE // K):
            pltpu.sync_copy(zbuf, acc.at[pl.ds(s * ROWS_PER_TILE + z * K, K)])
        plsc.subcore_barrier()

        col_off = j * N

        def blk(b, carry):
            base = s * per_tile + b * K
            pltpu.sync_copy(cols.at[pl.ds(base, K)], idxc)
            pltpu.sync_copy(rows.at[pl.ds(base, K)], idxr)
            pltpu.sync_copy(vals.at[pl.ds(base, K)], valv)
            for t in range(K // 16):
                idxc[pl.ds(t * 16, 16)] = idxc[pl.ds(t * 16, 16)] + col_off
            pltpu.async_copy(xc.at[idxc], gbuf, sem).wait()

            def rowscale(g, rc):
                v16 = valv[pl.ds(g * 16, 16)]
                for l in range(16):
                    vl = v16[l]
                    i = g * 16 + l
                    for w in range(CW // 16):
                        gbuf[i, pl.ds(w * 16, 16)] = (
                            gbuf[i, pl.ds(w * 16, 16)] * vl)
                return rc

            lax.fori_loop(0, K // 16, rowscale, 0)
            pltpu.sync_copy(gbuf, acc.at[idxr], add=True)
            return carry

        lax.fori_loop(0, nblk, blk, 0)
        plsc.subcore_barrier()
        pltpu.sync_copy(acc.at[pl.ds(s * ROWS_PER_TILE, ROWS_PER_TILE)],
                        out.at[j, pl.ds(s * ROWS_PER_TILE, ROWS_PER_TILE)])
        plsc.subcore_barrier()


def kernel(X, S_rows, S_cols, S_vals):
    nnz = S_rows.shape[0]
    per_tile = -(-nnz // NSUB)
    per_tile = -(-per_tile // K) * K
    nblk = per_tile // K
    pad = per_tile * NSUB - nnz
    rows_p = jnp.pad(S_rows, (0, pad))
    cols_p = jnp.pad(S_cols, (0, pad))
    vals_p = jnp.pad(S_vals, (0, pad))
    xc = X.reshape(NCHUNK, CW, N).transpose(0, 2, 1).reshape(NCHUNK * N, CW)

    mesh = plsc.VectorSubcoreMesh(core_axis_name="c", subcore_axis_name="s",
                                  num_cores=NCORES, num_subcores=NSUB)
    f = pl.kernel(
        functools.partial(_spmm_body, nblk),
        out_type=jax.ShapeDtypeStruct((NCHUNK, N, CW), jnp.float32),
        mesh=mesh,
        scratch_types=[
            pltpu.VMEM((K,), jnp.int32),
            pltpu.VMEM((K,), jnp.int32),
            pltpu.VMEM((K,), jnp.float32),
            pltpu.VMEM((K, CW), jnp.float32),
            pltpu.VMEM((K, CW), jnp.float32),
            pltpu.VMEM_SHARED((N, CW), jnp.float32),
            pltpu.SemaphoreType.DMA,
        ],
        compiler_params=pltpu.CompilerParams(use_tc_tiling_on_sc=False),
    )
    out_c = f(xc, rows_p, cols_p, vals_p)
    return out_c.transpose(0, 2, 1).reshape(BATCH, N)


# speedup vs baseline: 3.5197x; 1.1784x over previous
import functools

import jax
import jax.numpy as jnp
from jax import lax
from jax.experimental import pallas as pl
from jax.experimental.pallas import tpu as pltpu
from jax.experimental.pallas import tpu_sc as plsc

N = 16384
BATCH = 256
NCHUNK = 8
CW = BATCH // NCHUNK
NCORES = 2
NSUB = 16
CPC = NCHUNK // NCORES
K = 128
ROWS_PER_TILE = N // NSUB


def _spmm_body(nblk, xc, rows, cols, vals, out, idxc2, idxr2, valv2, gbuf,
               zbuf, acc, sem):
    c = lax.axis_index("c")
    s = lax.axis_index("s")
    per_tile = nblk * K

    zero16 = jnp.zeros((16,), jnp.float32)

    def zrow(i, carry):
        for w in range(CW // 16):
            zbuf[i, pl.ds(w * 16, 16)] = zero16
        return carry

    lax.fori_loop(0, K, zrow, 0)

    pltpu.sync_copy(rows.at[s], idxr2)
    pltpu.sync_copy(vals.at[s], valv2)

    for jl in range(CPC):
        j = c * CPC + jl
        for z in range(ROWS_PER_TILE // K):
            pltpu.sync_copy(zbuf, acc.at[pl.ds(s * ROWS_PER_TILE + z * K, K)])

        pltpu.sync_copy(cols.at[s], idxc2)
        col_off = j * N

        def adj(i, carry):
            idxc2[i // 8, pl.ds((i % 8) * 16, 16)] = (
                idxc2[i // 8, pl.ds((i % 8) * 16, 16)] + col_off)
            return carry

        lax.fori_loop(0, nblk * (K // 16), adj, 0)
        plsc.subcore_barrier()

        def blk(b, carry):
            pltpu.async_copy(xc.at[idxc2.at[b]], gbuf, sem).wait()

            def rowscale(g, rc):
                v16 = valv2[b, pl.ds(g * 16, 16)]
                for l in range(16):
                    vl = v16[l]
                    i = g * 16 + l
                    for w in range(CW // 16):
                        gbuf[i, pl.ds(w * 16, 16)] = (
                            gbuf[i, pl.ds(w * 16, 16)] * vl)
                return rc

            lax.fori_loop(0, K // 16, rowscale, 0)
            pltpu.sync_copy(gbuf, acc.at[idxr2.at[b]], add=True)
            return carry

        lax.fori_loop(0, nblk, blk, 0)
        plsc.subcore_barrier()
        pltpu.sync_copy(acc.at[pl.ds(s * ROWS_PER_TILE, ROWS_PER_TILE)],
                        out.at[j, pl.ds(s * ROWS_PER_TILE, ROWS_PER_TILE)])
        plsc.subcore_barrier()


def kernel(X, S_rows, S_cols, S_vals):
    nnz = S_rows.shape[0]
    per_tile = -(-nnz // NSUB)
    per_tile = -(-per_tile // K) * K
    nblk = per_tile // K
    pad = per_tile * NSUB - nnz
    rows_p = jnp.pad(S_rows, (0, pad)).reshape(NSUB, nblk, K)
    cols_p = jnp.pad(S_cols, (0, pad)).reshape(NSUB, nblk, K)
    vals_p = jnp.pad(S_vals, (0, pad)).reshape(NSUB, nblk, K)
    xc = X.reshape(NCHUNK, CW, N).transpose(0, 2, 1).reshape(NCHUNK * N, CW)

    mesh = plsc.VectorSubcoreMesh(core_axis_name="c", subcore_axis_name="s",
                                  num_cores=NCORES, num_subcores=NSUB)
    f = pl.kernel(
        functools.partial(_spmm_body, nblk),
        out_type=jax.ShapeDtypeStruct((NCHUNK, N, CW), jnp.float32),
        mesh=mesh,
        scratch_types=[
            pltpu.VMEM((nblk, K), jnp.int32),
            pltpu.VMEM((nblk, K), jnp.int32),
            pltpu.VMEM((nblk, K), jnp.float32),
            pltpu.VMEM((K, CW), jnp.float32),
            pltpu.VMEM((K, CW), jnp.float32),
            pltpu.VMEM_SHARED((N, CW), jnp.float32),
            pltpu.SemaphoreType.DMA,
        ],
        compiler_params=pltpu.CompilerParams(use_tc_tiling_on_sc=False),
    )
    out_c = f(xc, rows_p, cols_p, vals_p)
    return out_c.transpose(0, 2, 1).reshape(BATCH, N)


# speedup vs baseline: 5.4563x; 1.5502x over previous
import functools

import jax
import jax.numpy as jnp
from jax import lax
from jax.experimental import pallas as pl
from jax.experimental.pallas import tpu as pltpu
from jax.experimental.pallas import tpu_sc as plsc

N = 16384
BATCH = 256
NCHUNK = 8
CW = BATCH // NCHUNK
NCORES = 2
NSUB = 16
CPC = NCHUNK // NCORES
K = 128
ROWS_PER_TILE = N // NSUB


def _spmm_body(nblk, xc, rows, cols, vals, out, idxc2, idxr2, valv2, gbuf,
               gbuf1, zbuf, acc, sem, sem1):
    c = lax.axis_index("c")
    s = lax.axis_index("s")
    per_tile = nblk * K

    zero16 = jnp.zeros((16,), jnp.float32)

    def zrow(i, carry):
        for w in range(CW // 16):
            zbuf[i, pl.ds(w * 16, 16)] = zero16
        return carry

    lax.fori_loop(0, K, zrow, 0)

    pltpu.sync_copy(rows.at[s], idxr2)
    pltpu.sync_copy(vals.at[s], valv2)

    for jl in range(CPC):
        j = c * CPC + jl
        for z in range(ROWS_PER_TILE // K):
            pltpu.sync_copy(zbuf, acc.at[pl.ds(s * ROWS_PER_TILE + z * K, K)])

        pltpu.sync_copy(cols.at[s], idxc2)
        col_off = j * N

        def adj(i, carry):
            idxc2[i // 8, pl.ds((i % 8) * 16, 16)] = (
                idxc2[i // 8, pl.ds((i % 8) * 16, 16)] + col_off)
            return carry

        lax.fori_loop(0, nblk * (K // 16), adj, 0)
        plsc.subcore_barrier()

        def scale_scatter(b, buf):
            def rowscale(g, rc):
                v16 = valv2[b, pl.ds(g * 16, 16)]
                for l in range(16):
                    vl = v16[l]
                    i = g * 16 + l
                    for w in range(CW // 16):
                        buf[i, pl.ds(w * 16, 16)] = (
                            buf[i, pl.ds(w * 16, 16)] * vl)
                return rc

            lax.fori_loop(0, K // 16, rowscale, 0)
            pltpu.sync_copy(buf, acc.at[idxr2.at[b]], add=True)

        pltpu.async_copy(xc.at[idxc2.at[0]], gbuf, sem)

        def blk2(t, carry):
            b0 = 2 * t
            b1 = b0 + 1
            pltpu.async_copy(xc.at[idxc2.at[b1]], gbuf1, sem1)
            pltpu.make_async_copy(xc.at[idxc2.at[b0]], gbuf, sem).wait()
            scale_scatter(b0, gbuf)

            @pl.when(b1 + 1 < nblk)
            def _():
                pltpu.async_copy(xc.at[idxc2.at[b1 + 1]], gbuf, sem)

            pltpu.make_async_copy(xc.at[idxc2.at[b1]], gbuf1, sem1).wait()
            scale_scatter(b1, gbuf1)
            return carry

        lax.fori_loop(0, nblk // 2, blk2, 0)
        plsc.subcore_barrier()
        pltpu.sync_copy(acc.at[pl.ds(s * ROWS_PER_TILE, ROWS_PER_TILE)],
                        out.at[j, pl.ds(s * ROWS_PER_TILE, ROWS_PER_TILE)])
        plsc.subcore_barrier()


def kernel(X, S_rows, S_cols, S_vals):
    nnz = S_rows.shape[0]
    per_tile = -(-nnz // NSUB)
    per_tile = -(-per_tile // K) * K
    nblk = per_tile // K
    pad = per_tile * NSUB - nnz
    rows_p = jnp.pad(S_rows, (0, pad)).reshape(NSUB, nblk, K)
    cols_p = jnp.pad(S_cols, (0, pad)).reshape(NSUB, nblk, K)
    vals_p = jnp.pad(S_vals, (0, pad)).reshape(NSUB, nblk, K)
    xc = X.reshape(NCHUNK, CW, N).transpose(0, 2, 1).reshape(NCHUNK * N, CW)

    mesh = plsc.VectorSubcoreMesh(core_axis_name="c", subcore_axis_name="s",
                                  num_cores=NCORES, num_subcores=NSUB)
    f = pl.kernel(
        functools.partial(_spmm_body, nblk),
        out_type=jax.ShapeDtypeStruct((NCHUNK, N, CW), jnp.float32),
        mesh=mesh,
        scratch_types=[
            pltpu.VMEM((nblk, K), jnp.int32),
            pltpu.VMEM((nblk, K), jnp.int32),
            pltpu.VMEM((nblk, K), jnp.float32),
            pltpu.VMEM((K, CW), jnp.float32),
            pltpu.VMEM((K, CW), jnp.float32),
            pltpu.VMEM((K, CW), jnp.float32),
            pltpu.VMEM_SHARED((N, CW), jnp.float32),
            pltpu.SemaphoreType.DMA,
            pltpu.SemaphoreType.DMA,
        ],
        compiler_params=pltpu.CompilerParams(use_tc_tiling_on_sc=False),
    )
    out_c = f(xc, rows_p, cols_p, vals_p)
    return out_c.transpose(0, 2, 1).reshape(BATCH, N)


# speedup vs baseline: 5.8858x; 1.0787x over previous
import functools

import jax
import jax.numpy as jnp
from jax import lax
from jax.experimental import pallas as pl
from jax.experimental.pallas import tpu as pltpu
from jax.experimental.pallas import tpu_sc as plsc

N = 16384
BATCH = 256
NCHUNK = 8
CW = BATCH // NCHUNK
NCORES = 2
NSUB = 16
CPC = NCHUNK // NCORES
K = 128
ROWS_PER_TILE = N // NSUB


def _spmm_body(nblk, xc, rows, cols, vals, out, idxc2, idxr2, valv2, gbuf,
               gbuf1, gbuf2, zbuf, acc, sem, sem1, sem2, sems0, sems1, sems2):
    c = lax.axis_index("c")
    s = lax.axis_index("s")
    per_tile = nblk * K

    zero16 = jnp.zeros((16,), jnp.float32)

    def zrow(i, carry):
        for w in range(CW // 16):
            zbuf[i, pl.ds(w * 16, 16)] = zero16
        return carry

    lax.fori_loop(0, K, zrow, 0)

    pltpu.sync_copy(rows.at[s], idxr2)
    pltpu.sync_copy(vals.at[s], valv2)

    for jl in range(CPC):
        j = c * CPC + jl
        for z in range(ROWS_PER_TILE // K):
            pltpu.sync_copy(zbuf, acc.at[pl.ds(s * ROWS_PER_TILE + z * K, K)])

        pltpu.sync_copy(cols.at[s], idxc2)
        col_off = j * N

        def adj(i, carry):
            idxc2[i // 8, pl.ds((i % 8) * 16, 16)] = (
                idxc2[i // 8, pl.ds((i % 8) * 16, 16)] + col_off)
            return carry

        lax.fori_loop(0, nblk * (K // 16), adj, 0)
        plsc.subcore_barrier()

        def scale(b, buf):
            def rowscale(g, rc):
                v16 = valv2[b, pl.ds(g * 16, 16)]
                for l in range(16):
                    vl = v16[l]
                    i = g * 16 + l
                    for w in range(CW // 16):
                        buf[i, pl.ds(w * 16, 16)] = (
                            buf[i, pl.ds(w * 16, 16)] * vl)
                return rc

            lax.fori_loop(0, K // 16, rowscale, 0)

        bufs = (gbuf, gbuf1, gbuf2)
        gsem = (sem, sem1, sem2)
        ssem = (sems0, sems1, sems2)
        pltpu.async_copy(xc.at[idxc2.at[0]], bufs[0], gsem[0])
        pltpu.async_copy(xc.at[idxc2.at[1]], bufs[1], gsem[1])

        def blk3(t, carry):
            for u in range(3):
                b = 3 * t + u
                up = (u + 2) % 3
                pltpu.make_async_copy(xc.at[idxc2.at[b]], bufs[u],
                                      gsem[u]).wait()
                scale(b, bufs[u])
                pltpu.async_copy(bufs[u], acc.at[idxr2.at[b]], ssem[u],
                                 add=True)

                @pl.when(b >= 1)
                def _():
                    pltpu.make_async_copy(bufs[up], acc.at[idxr2.at[b - 1]],
                                          ssem[up]).wait()

                @pl.when(b + 2 < nblk)
                def _():
                    pltpu.async_copy(xc.at[idxc2.at[b + 2]], bufs[up],
                                     gsem[up])
            return carry

        lax.fori_loop(0, nblk // 3, blk3, 0)
        pltpu.make_async_copy(bufs[(nblk - 1) % 3],
                              acc.at[idxr2.at[nblk - 1]],
                              ssem[(nblk - 1) % 3]).wait()
        plsc.subcore_barrier()
        pltpu.sync_copy(acc.at[pl.ds(s * ROWS_PER_TILE, ROWS_PER_TILE)],
                        out.at[j, pl.ds(s * ROWS_PER_TILE, ROWS_PER_TILE)])
        plsc.subcore_barrier()


def kernel(X, S_rows, S_cols, S_vals):
    nnz = S_rows.shape[0]
    per_tile = -(-nnz // NSUB)
    per_tile = -(-per_tile // (3 * K)) * (3 * K)
    nblk = per_tile // K
    pad = per_tile * NSUB - nnz
    rows_p = jnp.pad(S_rows, (0, pad)).reshape(NSUB, nblk, K)
    cols_p = jnp.pad(S_cols, (0, pad)).reshape(NSUB, nblk, K)
    vals_p = jnp.pad(S_vals, (0, pad)).reshape(NSUB, nblk, K)
    xc = X.reshape(NCHUNK, CW, N).transpose(0, 2, 1).reshape(NCHUNK * N, CW)

    mesh = plsc.VectorSubcoreMesh(core_axis_name="c", subcore_axis_name="s",
                                  num_cores=NCORES, num_subcores=NSUB)
    f = pl.kernel(
        functools.partial(_spmm_body, nblk),
        out_type=jax.ShapeDtypeStruct((NCHUNK, N, CW), jnp.float32),
        mesh=mesh,
        scratch_types=[
            pltpu.VMEM((nblk, K), jnp.int32),
            pltpu.VMEM((nblk, K), jnp.int32),
            pltpu.VMEM((nblk, K), jnp.float32),
            pltpu.VMEM((K, CW), jnp.float32),
            pltpu.VMEM((K, CW), jnp.float32),
            pltpu.VMEM((K, CW), jnp.float32),
            pltpu.VMEM((K, CW), jnp.float32),
            pltpu.VMEM_SHARED((N, CW), jnp.float32),
            pltpu.SemaphoreType.DMA,
            pltpu.SemaphoreType.DMA,
            pltpu.SemaphoreType.DMA,
            pltpu.SemaphoreType.DMA,
            pltpu.SemaphoreType.DMA,
            pltpu.SemaphoreType.DMA,
        ],
        compiler_params=pltpu.CompilerParams(use_tc_tiling_on_sc=False),
    )
    out_c = f(xc, rows_p, cols_p, vals_p)
    return out_c.transpose(0, 2, 1).reshape(BATCH, N)
